# Initial kernel scaffold; baseline (speedup 1.0000x reference)
import functools

import numpy as np
import jax
import jax.numpy as jnp
from jax import lax
from jax.experimental import pallas as pl
from jax.experimental.pallas import tpu as pltpu
from jax.experimental.pallas import tpu_sc as plsc

N = 10000
D = 128
E = 320000
NC = 2
NS = 16
NW = NC * NS
EP = 327680
ET = EP // NW
K = 128
NCHUNK = ET // K
N2 = 10240
STRIPE = N2 // NS
EXP1 = float(np.exp(np.float32(1.0)))

_MESH = dict(core_axis_name="c", subcore_axis_name="s")


@functools.partial(
    pl.kernel,
    mesh=plsc.VectorSubcoreMesh(**_MESH),
    out_type=(
        jax.ShapeDtypeStruct((NC, N2), jnp.float32),
        jax.ShapeDtypeStruct((NC, N2), jnp.float32),
    ),
    scratch_types=[
        pltpu.VMEM((K,), jnp.int32),
        pltpu.VMEM((K,), jnp.float32),
        pltpu.VMEM((K,), jnp.float32),
        pltpu.VMEM((K,), jnp.float32),
        pltpu.VMEM((STRIPE,), jnp.float32),
        pltpu.VMEM_SHARED((N2,), jnp.float32),
        pltpu.VMEM_SHARED((N2,), jnp.float32),
    ],
)
def _scalar_pass(dst_hbm, ew_hbm, d_out, c_out,
                 dstv, ewv, valsv, onesv, zv, d_sh, c_sh):
    c = lax.axis_index("c")
    s = lax.axis_index("s")

    def zbody(j, _):
        zv[pl.ds(j * 16, 16)] = jnp.zeros((16,), jnp.float32)
        return 0
    lax.fori_loop(0, STRIPE // 16, zbody, 0)
    pltpu.sync_copy(zv, d_sh.at[pl.ds(s * STRIPE, STRIPE)])
    pltpu.sync_copy(zv, c_sh.at[pl.ds(s * STRIPE, STRIPE)])
    plsc.subcore_barrier()

    base0 = c * (EP // NC) + s * ET

    def chunk(i, _):
        base = base0 + i * K
        pltpu.sync_copy(ew_hbm.at[pl.ds(base, K)], ewv)
        pltpu.sync_copy(dst_hbm.at[pl.ds(base, K)], dstv)

        def compute(j, _):
            sl = pl.ds(j * 16, 16)
            w16 = ewv[sl]
            valsv[sl] = jnp.exp(w16)
            onesv[sl] = jnp.where(w16 > jnp.float32(-50.0),
                                  jnp.float32(1.0), jnp.float32(0.0))
            return 0
        lax.fori_loop(0, K // 16, compute, 0)
        pltpu.sync_copy(valsv, d_sh.at[dstv], add=True)
        pltpu.sync_copy(onesv, c_sh.at[dstv], add=True)
        return 0
    lax.fori_loop(0, NCHUNK, chunk, 0)
    plsc.subcore_barrier()
    pltpu.sync_copy(d_sh.at[pl.ds(s * STRIPE, STRIPE)],
                    d_out.at[c, pl.ds(s * STRIPE, STRIPE)])
    pltpu.sync_copy(c_sh.at[pl.ds(s * STRIPE, STRIPE)],
                    c_out.at[c, pl.ds(s * STRIPE, STRIPE)])


@functools.partial(
    pl.kernel,
    mesh=plsc.VectorSubcoreMesh(**_MESH),
    out_type=(
        jax.ShapeDtypeStruct((EP,), jnp.float32),
        jax.ShapeDtypeStruct((N2,), jnp.float32),
    ),
    scratch_types=[
        pltpu.VMEM((N2,), jnp.float32),
        pltpu.VMEM((N2,), jnp.float32),
        pltpu.VMEM((N2,), jnp.float32),
        pltpu.VMEM((N2 // NW,), jnp.float32),
        pltpu.VMEM((K,), jnp.int32),
        pltpu.VMEM((K,), jnp.float32),
        pltpu.VMEM((K,), jnp.float32),
    ],
)
def _coeff_pass(dst_hbm, ew_hbm, dpart, cpart, coeff_out, selfc_out,
                dloc, cloc, tmp, selfv, dstv, ewv, outv):
    c = lax.axis_index("c")
    s = lax.axis_index("s")
    wid = s * NC + c

    pltpu.sync_copy(dpart.at[0], dloc)
    pltpu.sync_copy(dpart.at[1], tmp)

    def comb_d(j, _):
        sl = pl.ds(j * 16, 16)
        dloc[sl] = dloc[sl] + tmp[sl] + jnp.float32(EXP1)
        return 0
    lax.fori_loop(0, N2 // 16, comb_d, 0)

    pltpu.sync_copy(cpart.at[0], cloc)
    pltpu.sync_copy(cpart.at[1], tmp)

    def comb_c(j, _):
        sl = pl.ds(j * 16, 16)
        cloc[sl] = cloc[sl] + tmp[sl] + jnp.float32(1.0)
        return 0
    lax.fori_loop(0, N2 // 16, comb_c, 0)

    nper = N2 // NW
    nbase = wid * nper

    def selfc_body(j, _):
        sl = pl.ds(j * 16, 16)
        gl = pl.ds(nbase + j * 16, 16)
        selfv[sl] = jnp.float32(EXP1) / (dloc[gl] * cloc[gl])
        return 0
    lax.fori_loop(0, nper // 16, selfc_body, 0)
    pltpu.sync_copy(selfv, selfc_out.at[pl.ds(nbase, nper)])

    base0 = c * (EP // NC) + s * ET

    def chunk(i, _):
        base = base0 + i * K
        pltpu.sync_copy(ew_hbm.at[pl.ds(base, K)], ewv)
        pltpu.sync_copy(dst_hbm.at[pl.ds(base, K)], dstv)

        def compute(j, _):
            sl = pl.ds(j * 16, 16)
            idx = dstv[sl]
            dd = plsc.load_gather(dloc, [idx])
            cc = plsc.load_gather(cloc, [idx])
            outv[sl] = jnp.exp(ewv[sl]) / (dd * cc)
            return 0
        lax.fori_loop(0, K // 16, compute, 0)
        pltpu.sync_copy(outv, coeff_out.at[pl.ds(base, K)])
        return 0
    lax.fori_loop(0, NCHUNK, chunk, 0)


@functools.partial(
    pl.kernel,
    mesh=plsc.VectorSubcoreMesh(**_MESH),
    out_type=jax.ShapeDtypeStruct((NC, N2, D), jnp.float32),
    scratch_types=[
        pltpu.VMEM((K,), jnp.int32),
        pltpu.VMEM((K,), jnp.int32),
        pltpu.VMEM((K,), jnp.float32),
        pltpu.VMEM((K, D), jnp.float32),
        pltpu.VMEM_SHARED((N2, D), jnp.float32),
        pltpu.SemaphoreType.DMA,
    ],
)
def _row_pass(src_hbm, dst_hbm, coeff_hbm, xl_hbm, acc_out,
              srcv, dstv, cfv, rows, acc_sh, sem):
    c = lax.axis_index("c")
    s = lax.axis_index("s")

    def zrow(j, _):
        for q in range(D // 16):
            rows[j, pl.ds(q * 16, 16)] = jnp.zeros((16,), jnp.float32)
        return 0
    lax.fori_loop(0, K, zrow, 0)
    for z in range(STRIPE // K):
        pltpu.sync_copy(rows, acc_sh.at[pl.ds(s * STRIPE + z * K, K)])
    plsc.subcore_barrier()

    base0 = c * (EP // NC) + s * ET

    def chunk(i, _):
        base = base0 + i * K
        pltpu.sync_copy(src_hbm.at[pl.ds(base, K)], srcv)
        pltpu.sync_copy(coeff_hbm.at[pl.ds(base, K)], cfv)
        pltpu.sync_copy(dst_hbm.at[pl.ds(base, K)], dstv)
        pltpu.async_copy(xl_hbm.at[srcv], rows, sem).wait()

        def scale(j, _):
            cb = plsc.load_gather(cfv, [jnp.full((16,), j, jnp.int32)])
            for q in range(D // 16):
                sl = pl.ds(q * 16, 16)
                rows[j, sl] = rows[j, sl] * cb
            return 0
        lax.fori_loop(0, K, scale, 0)
        pltpu.sync_copy(rows, acc_sh.at[dstv], add=True)
        return 0
    lax.fori_loop(0, NCHUNK, chunk, 0)
    plsc.subcore_barrier()
    pltpu.sync_copy(acc_sh.at[pl.ds(s * STRIPE, STRIPE)],
                    acc_out.at[c, pl.ds(s * STRIPE, STRIPE)])


def _mm_bias(xin, w, b):
    m = xin.shape[0]
    bm = 1000

    def body(x_ref, w_ref, b_ref, o_ref):
        o_ref[...] = jnp.dot(x_ref[...], w_ref[...],
                             preferred_element_type=jnp.float32) + b_ref[...]

    return pl.pallas_call(
        body,
        grid=(m // bm,),
        in_specs=[
            pl.BlockSpec((bm, D), lambda i: (i, 0)),
            pl.BlockSpec((D, D), lambda i: (0, 0)),
            pl.BlockSpec((1, D), lambda i: (0, 0)),
        ],
        out_specs=pl.BlockSpec((bm, D), lambda i: (i, 0)),
        out_shape=jax.ShapeDtypeStruct((m, D), jnp.float32),
    )(xin, w, b.reshape(1, D))


def _post(acc0, acc1, selfc, xl, w, relu):
    m = xl.shape[0]
    bm = 1000

    def body(a0_ref, a1_ref, sc_ref, x_ref, w_ref, o_ref):
        aggr = a0_ref[...] + a1_ref[...] + sc_ref[...] * x_ref[...]
        o = jnp.dot(aggr, w_ref[...], preferred_element_type=jnp.float32)
        if relu:
            o = jnp.maximum(o, jnp.float32(0.0))
        o_ref[...] = o

    return pl.pallas_call(
        body,
        grid=(m // bm,),
        in_specs=[
            pl.BlockSpec((bm, D), lambda i: (i, 0)),
            pl.BlockSpec((bm, D), lambda i: (i, 0)),
            pl.BlockSpec((bm, 1), lambda i: (i, 0)),
            pl.BlockSpec((bm, D), lambda i: (i, 0)),
            pl.BlockSpec((D, D), lambda i: (0, 0)),
        ],
        out_specs=pl.BlockSpec((bm, D), lambda i: (i, 0)),
        out_shape=jax.ShapeDtypeStruct((m, D), jnp.float32),
    )(acc0, acc1, selfc, xl, w)


def kernel(x, edge_index, edge_weight, lin1_w, lin1_b, w1, lin2_w, lin2_b, w2):
    src = edge_index[0]
    dst = edge_index[1]
    pad = EP - E
    zpad = jnp.zeros((pad,), jnp.int32)
    src_p = jnp.concatenate([src, zpad])
    dst_p = jnp.concatenate([dst, zpad])
    ew_p = jnp.concatenate([edge_weight,
                            jnp.full((pad,), -100.0, jnp.float32)])

    dpart, cpart = _scalar_pass(dst_p, ew_p)
    coeff, selfc = _coeff_pass(dst_p, ew_p, dpart, cpart)
    selfc = selfc[:N].reshape(N, 1)

    xl1 = _mm_bias(x, lin1_w, lin1_b)
    acc1 = _row_pass(src_p, dst_p, coeff, xl1)
    h = _post(acc1[0, :N], acc1[1, :N], selfc, xl1, w1, relu=True)

    xl2 = _mm_bias(h, lin2_w, lin2_b)
    acc2 = _row_pass(src_p, dst_p, coeff, xl2)
    out = _post(acc2[0, :N], acc2[1, :N], selfc, xl2, w2, relu=False)
    return out


# speedup vs baseline: 5.7525x; 5.7525x over previous
import functools

import numpy as np
import jax
import jax.numpy as jnp
from jax import lax
from jax.experimental import pallas as pl
from jax.experimental.pallas import tpu as pltpu
from jax.experimental.pallas import tpu_sc as plsc

N = 10000
D = 128
E = 320000
NC = 2
NS = 16
NW = NC * NS
EP = 327680
ET = EP // NW
K = 128
NCHUNK = ET // K
N2 = 10240
STRIPE = N2 // NS
EXP1 = float(np.exp(np.float32(1.0)))

_MESH = dict(core_axis_name="c", subcore_axis_name="s")


@functools.partial(
    pl.kernel,
    mesh=plsc.VectorSubcoreMesh(**_MESH),
    compiler_params=pltpu.CompilerParams(needs_layout_passes=False),
    out_type=(
        jax.ShapeDtypeStruct((NC, N2), jnp.float32),
        jax.ShapeDtypeStruct((NC, N2), jnp.float32),
    ),
    scratch_types=[
        pltpu.VMEM((K,), jnp.int32),
        pltpu.VMEM((K,), jnp.float32),
        pltpu.VMEM((K,), jnp.float32),
        pltpu.VMEM((K,), jnp.float32),
        pltpu.VMEM((STRIPE,), jnp.float32),
        pltpu.VMEM_SHARED((N2,), jnp.float32),
        pltpu.VMEM_SHARED((N2,), jnp.float32),
    ],
)
def _scalar_pass(dst_hbm, ew_hbm, d_out, c_out,
                 dstv, ewv, valsv, onesv, zv, d_sh, c_sh):
    c = lax.axis_index("c")
    s = lax.axis_index("s")

    def zbody(j, _):
        zv[pl.ds(j * 16, 16)] = jnp.zeros((16,), jnp.float32)
        return 0
    lax.fori_loop(0, STRIPE // 16, zbody, 0)
    pltpu.sync_copy(zv, d_sh.at[pl.ds(s * STRIPE, STRIPE)])
    pltpu.sync_copy(zv, c_sh.at[pl.ds(s * STRIPE, STRIPE)])
    plsc.subcore_barrier()

    base0 = c * (EP // NC) + s * ET

    def chunk(i, _):
        base = base0 + i * K
        pltpu.sync_copy(ew_hbm.at[pl.ds(base, K)], ewv)
        pltpu.sync_copy(dst_hbm.at[pl.ds(base, K)], dstv)

        def compute(j, _):
            sl = pl.ds(j * 16, 16)
            w16 = ewv[sl]
            valsv[sl] = jnp.exp(w16)
            onesv[sl] = jnp.where(w16 > jnp.float32(-50.0),
                                  jnp.float32(1.0), jnp.float32(0.0))
            return 0
        lax.fori_loop(0, K // 16, compute, 0)
        pltpu.sync_copy(valsv, d_sh.at[dstv], add=True)
        pltpu.sync_copy(onesv, c_sh.at[dstv], add=True)
        return 0
    lax.fori_loop(0, NCHUNK, chunk, 0)
    plsc.subcore_barrier()
    pltpu.sync_copy(d_sh.at[pl.ds(s * STRIPE, STRIPE)],
                    d_out.at[c, pl.ds(s * STRIPE, STRIPE)])
    pltpu.sync_copy(c_sh.at[pl.ds(s * STRIPE, STRIPE)],
                    c_out.at[c, pl.ds(s * STRIPE, STRIPE)])


@functools.partial(
    pl.kernel,
    mesh=plsc.VectorSubcoreMesh(**_MESH),
    compiler_params=pltpu.CompilerParams(needs_layout_passes=False),
    out_type=(
        jax.ShapeDtypeStruct((EP,), jnp.float32),
        jax.ShapeDtypeStruct((N2,), jnp.float32),
    ),
    scratch_types=[
        pltpu.VMEM((N2,), jnp.float32),
        pltpu.VMEM((N2,), jnp.float32),
        pltpu.VMEM((N2,), jnp.float32),
        pltpu.VMEM((N2 // NW,), jnp.float32),
        pltpu.VMEM((K,), jnp.int32),
        pltpu.VMEM((K,), jnp.float32),
        pltpu.VMEM((K,), jnp.float32),
    ],
)
def _coeff_pass(dst_hbm, ew_hbm, dpart, cpart, coeff_out, selfc_out,
                dloc, cloc, tmp, selfv, dstv, ewv, outv):
    c = lax.axis_index("c")
    s = lax.axis_index("s")
    wid = s * NC + c

    pltpu.sync_copy(dpart.at[0], dloc)
    pltpu.sync_copy(dpart.at[1], tmp)

    def comb_d(j, _):
        sl = pl.ds(j * 16, 16)
        dloc[sl] = dloc[sl] + tmp[sl] + jnp.float32(EXP1)
        return 0
    lax.fori_loop(0, N2 // 16, comb_d, 0)

    pltpu.sync_copy(cpart.at[0], cloc)
    pltpu.sync_copy(cpart.at[1], tmp)

    def comb_c(j, _):
        sl = pl.ds(j * 16, 16)
        cloc[sl] = cloc[sl] + tmp[sl] + jnp.float32(1.0)
        return 0
    lax.fori_loop(0, N2 // 16, comb_c, 0)

    nper = N2 // NW
    nbase = wid * nper

    def selfc_body(j, _):
        sl = pl.ds(j * 16, 16)
        gl = pl.ds(nbase + j * 16, 16)
        selfv[sl] = jnp.float32(EXP1) / (dloc[gl] * cloc[gl])
        return 0
    lax.fori_loop(0, nper // 16, selfc_body, 0)
    pltpu.sync_copy(selfv, selfc_out.at[pl.ds(nbase, nper)])

    base0 = c * (EP // NC) + s * ET

    def chunk(i, _):
        base = base0 + i * K
        pltpu.sync_copy(ew_hbm.at[pl.ds(base, K)], ewv)
        pltpu.sync_copy(dst_hbm.at[pl.ds(base, K)], dstv)

        def compute(j, _):
            sl = pl.ds(j * 16, 16)
            idx = dstv[sl]
            dd = plsc.load_gather(dloc, [idx])
            cc = plsc.load_gather(cloc, [idx])
            outv[sl] = jnp.exp(ewv[sl]) / (dd * cc)
            return 0
        lax.fori_loop(0, K // 16, compute, 0)
        pltpu.sync_copy(outv, coeff_out.at[pl.ds(base, K)])
        return 0
    lax.fori_loop(0, NCHUNK, chunk, 0)


@functools.partial(
    pl.kernel,
    mesh=plsc.VectorSubcoreMesh(**_MESH),
    compiler_params=pltpu.CompilerParams(needs_layout_passes=False),
    out_type=jax.ShapeDtypeStruct((NC, N2, D), jnp.float32),
    scratch_types=[
        pltpu.VMEM((K,), jnp.int32),
        pltpu.VMEM((K,), jnp.int32),
        pltpu.VMEM((K,), jnp.float32),
        pltpu.VMEM((K, D), jnp.float32),
        pltpu.VMEM_SHARED((N2, D), jnp.float32),
        pltpu.SemaphoreType.DMA,
    ],
)
def _row_pass(src_hbm, dst_hbm, coeff_hbm, xl_hbm, acc_out,
              srcv, dstv, cfv, rows, acc_sh, sem):
    c = lax.axis_index("c")
    s = lax.axis_index("s")

    def zrow(j, _):
        for q in range(D // 16):
            rows[j, pl.ds(q * 16, 16)] = jnp.zeros((16,), jnp.float32)
        return 0
    lax.fori_loop(0, K, zrow, 0)
    for z in range(STRIPE // K):
        pltpu.sync_copy(rows, acc_sh.at[pl.ds(s * STRIPE + z * K, K)])
    plsc.subcore_barrier()

    base0 = c * (EP // NC) + s * ET

    def chunk(i, _):
        base = base0 + i * K
        pltpu.sync_copy(src_hbm.at[pl.ds(base, K)], srcv)
        pltpu.sync_copy(coeff_hbm.at[pl.ds(base, K)], cfv)
        pltpu.sync_copy(dst_hbm.at[pl.ds(base, K)], dstv)
        pltpu.async_copy(xl_hbm.at[srcv], rows, sem).wait()

        def scale(j, _):
            cb = plsc.load_gather(cfv, [jnp.full((16,), j, jnp.int32)])
            for q in range(D // 16):
                sl = pl.ds(q * 16, 16)
                rows[j, sl] = rows[j, sl] * cb
            return 0
        lax.fori_loop(0, K, scale, 0)
        pltpu.sync_copy(rows, acc_sh.at[dstv], add=True)
        return 0
    lax.fori_loop(0, NCHUNK, chunk, 0)
    plsc.subcore_barrier()
    pltpu.sync_copy(acc_sh.at[pl.ds(s * STRIPE, STRIPE)],
                    acc_out.at[c, pl.ds(s * STRIPE, STRIPE)])


def _mm_bias(xin, w, b):
    m = xin.shape[0]
    bm = 1000

    def body(x_ref, w_ref, b_ref, o_ref):
        o_ref[...] = jnp.dot(x_ref[...], w_ref[...],
                             preferred_element_type=jnp.float32) + b_ref[...]

    return pl.pallas_call(
        body,
        grid=(m // bm,),
        in_specs=[
            pl.BlockSpec((bm, D), lambda i: (i, 0)),
            pl.BlockSpec((D, D), lambda i: (0, 0)),
            pl.BlockSpec((1, D), lambda i: (0, 0)),
        ],
        out_specs=pl.BlockSpec((bm, D), lambda i: (i, 0)),
        out_shape=jax.ShapeDtypeStruct((m, D), jnp.float32),
    )(xin, w, b.reshape(1, D))


def _post(acc0, acc1, selfc, xl, w, relu):
    m = xl.shape[0]
    bm = 1000

    def body(a0_ref, a1_ref, sc_ref, x_ref, w_ref, o_ref):
        aggr = a0_ref[...] + a1_ref[...] + sc_ref[...] * x_ref[...]
        o = jnp.dot(aggr, w_ref[...], preferred_element_type=jnp.float32)
        if relu:
            o = jnp.maximum(o, jnp.float32(0.0))
        o_ref[...] = o

    return pl.pallas_call(
        body,
        grid=(m // bm,),
        in_specs=[
            pl.BlockSpec((bm, D), lambda i: (i, 0)),
            pl.BlockSpec((bm, D), lambda i: (i, 0)),
            pl.BlockSpec((bm, 1), lambda i: (i, 0)),
            pl.BlockSpec((bm, D), lambda i: (i, 0)),
            pl.BlockSpec((D, D), lambda i: (0, 0)),
        ],
        out_specs=pl.BlockSpec((bm, D), lambda i: (i, 0)),
        out_shape=jax.ShapeDtypeStruct((m, D), jnp.float32),
    )(acc0, acc1, selfc, xl, w)


def kernel(x, edge_index, edge_weight, lin1_w, lin1_b, w1, lin2_w, lin2_b, w2):
    src = edge_index[0]
    dst = edge_index[1]
    pad = EP - E
    zpad = jnp.zeros((pad,), jnp.int32)
    src_p = jnp.concatenate([src, zpad])
    dst_p = jnp.concatenate([dst, zpad])
    ew_p = jnp.concatenate([edge_weight,
                            jnp.full((pad,), -100.0, jnp.float32)])

    dpart, cpart = _scalar_pass(dst_p, ew_p)
    coeff, selfc = _coeff_pass(dst_p, ew_p, dpart, cpart)
    selfc = selfc[:N].reshape(N, 1)

    xl1 = _mm_bias(x, lin1_w, lin1_b)
    acc1 = _row_pass(src_p, dst_p, coeff, xl1)
    h = _post(acc1[0, :N], acc1[1, :N], selfc, xl1, w1, relu=True)

    xl2 = _mm_bias(h, lin2_w, lin2_b)
    acc2 = _row_pass(src_p, dst_p, coeff, xl2)
    out = _post(acc2[0, :N], acc2[1, :N], selfc, xl2, w2, relu=False)
    return out


# speedup vs baseline: 7.2214x; 1.2554x over previous
import functools

import numpy as np
import jax
import jax.numpy as jnp
from jax import lax
from jax.experimental import pallas as pl
from jax.experimental.pallas import tpu as pltpu
from jax.experimental.pallas import tpu_sc as plsc

N = 10000
D = 128
E = 320000
NC = 2
NS = 16
NW = NC * NS
EP = 327680
ET = EP // NW
K = 64
NCHUNK = ET // K
N2 = 10240
STRIPE = N2 // NS
EXP1 = float(np.exp(np.float32(1.0)))

_MESH = dict(core_axis_name="c", subcore_axis_name="s")


@functools.partial(
    pl.kernel,
    mesh=plsc.VectorSubcoreMesh(**_MESH),
    compiler_params=pltpu.CompilerParams(needs_layout_passes=False),
    out_type=(
        jax.ShapeDtypeStruct((NC, N2), jnp.float32),
        jax.ShapeDtypeStruct((NC, N2), jnp.float32),
    ),
    scratch_types=[
        pltpu.VMEM((K,), jnp.int32),
        pltpu.VMEM((K,), jnp.float32),
        pltpu.VMEM((K,), jnp.float32),
        pltpu.VMEM((K,), jnp.float32),
        pltpu.VMEM((STRIPE,), jnp.float32),
        pltpu.VMEM_SHARED((N2,), jnp.float32),
        pltpu.VMEM_SHARED((N2,), jnp.float32),
    ],
)
def _scalar_pass(dst_hbm, ew_hbm, d_out, c_out,
                 dstv, ewv, valsv, onesv, zv, d_sh, c_sh):
    c = lax.axis_index("c")
    s = lax.axis_index("s")

    def zbody(j, _):
        zv[pl.ds(j * 16, 16)] = jnp.zeros((16,), jnp.float32)
        return 0
    lax.fori_loop(0, STRIPE // 16, zbody, 0)
    pltpu.sync_copy(zv, d_sh.at[pl.ds(s * STRIPE, STRIPE)])
    pltpu.sync_copy(zv, c_sh.at[pl.ds(s * STRIPE, STRIPE)])
    plsc.subcore_barrier()

    base0 = c * (EP // NC) + s * ET

    def chunk(i, _):
        base = base0 + i * K
        pltpu.sync_copy(ew_hbm.at[pl.ds(base, K)], ewv)
        pltpu.sync_copy(dst_hbm.at[pl.ds(base, K)], dstv)

        def compute(j, _):
            sl = pl.ds(j * 16, 16)
            w16 = ewv[sl]
            valsv[sl] = jnp.exp(w16)
            onesv[sl] = jnp.where(w16 > jnp.float32(-50.0),
                                  jnp.float32(1.0), jnp.float32(0.0))
            return 0
        lax.fori_loop(0, K // 16, compute, 0)
        pltpu.sync_copy(valsv, d_sh.at[dstv], add=True)
        pltpu.sync_copy(onesv, c_sh.at[dstv], add=True)
        return 0
    lax.fori_loop(0, NCHUNK, chunk, 0)
    plsc.subcore_barrier()
    pltpu.sync_copy(d_sh.at[pl.ds(s * STRIPE, STRIPE)],
                    d_out.at[c, pl.ds(s * STRIPE, STRIPE)])
    pltpu.sync_copy(c_sh.at[pl.ds(s * STRIPE, STRIPE)],
                    c_out.at[c, pl.ds(s * STRIPE, STRIPE)])


@functools.partial(
    pl.kernel,
    mesh=plsc.VectorSubcoreMesh(**_MESH),
    compiler_params=pltpu.CompilerParams(needs_layout_passes=False),
    out_type=(
        jax.ShapeDtypeStruct((EP,), jnp.float32),
        jax.ShapeDtypeStruct((N2,), jnp.float32),
    ),
    scratch_types=[
        pltpu.VMEM((N2,), jnp.float32),
        pltpu.VMEM((N2,), jnp.float32),
        pltpu.VMEM((N2,), jnp.float32),
        pltpu.VMEM((N2 // NW,), jnp.float32),
        pltpu.VMEM((K,), jnp.int32),
        pltpu.VMEM((K,), jnp.float32),
        pltpu.VMEM((K,), jnp.float32),
    ],
)
def _coeff_pass(dst_hbm, ew_hbm, dpart, cpart, coeff_out, selfc_out,
                dloc, cloc, tmp, selfv, dstv, ewv, outv):
    c = lax.axis_index("c")
    s = lax.axis_index("s")
    wid = s * NC + c

    pltpu.sync_copy(dpart.at[0], dloc)
    pltpu.sync_copy(dpart.at[1], tmp)

    def comb_d(j, _):
        sl = pl.ds(j * 16, 16)
        dloc[sl] = dloc[sl] + tmp[sl] + jnp.float32(EXP1)
        return 0
    lax.fori_loop(0, N2 // 16, comb_d, 0)

    pltpu.sync_copy(cpart.at[0], cloc)
    pltpu.sync_copy(cpart.at[1], tmp)

    def comb_c(j, _):
        sl = pl.ds(j * 16, 16)
        cloc[sl] = cloc[sl] + tmp[sl] + jnp.float32(1.0)
        return 0
    lax.fori_loop(0, N2 // 16, comb_c, 0)

    nper = N2 // NW
    nbase = wid * nper

    def selfc_body(j, _):
        sl = pl.ds(j * 16, 16)
        gl = pl.ds(nbase + j * 16, 16)
        selfv[sl] = jnp.float32(EXP1) / (dloc[gl] * cloc[gl])
        return 0
    lax.fori_loop(0, nper // 16, selfc_body, 0)
    pltpu.sync_copy(selfv, selfc_out.at[pl.ds(nbase, nper)])

    base0 = c * (EP // NC) + s * ET

    def chunk(i, _):
        base = base0 + i * K
        pltpu.sync_copy(ew_hbm.at[pl.ds(base, K)], ewv)
        pltpu.sync_copy(dst_hbm.at[pl.ds(base, K)], dstv)

        def compute(j, _):
            sl = pl.ds(j * 16, 16)
            idx = dstv[sl]
            dd = plsc.load_gather(dloc, [idx])
            cc = plsc.load_gather(cloc, [idx])
            outv[sl] = jnp.exp(ewv[sl]) / (dd * cc)
            return 0
        lax.fori_loop(0, K // 16, compute, 0)
        pltpu.sync_copy(outv, coeff_out.at[pl.ds(base, K)])
        return 0
    lax.fori_loop(0, NCHUNK, chunk, 0)


NBUF = 4

@functools.partial(
    pl.kernel,
    mesh=plsc.VectorSubcoreMesh(**_MESH),
    compiler_params=pltpu.CompilerParams(needs_layout_passes=False),
    out_type=jax.ShapeDtypeStruct((NC, N2, D), jnp.float32),
    scratch_types=[
        [pltpu.VMEM((K,), jnp.int32) for _ in range(NBUF)],
        [pltpu.VMEM((K,), jnp.int32) for _ in range(NBUF)],
        [pltpu.VMEM((K,), jnp.float32) for _ in range(NBUF)],
        [pltpu.VMEM((K, D), jnp.float32) for _ in range(NBUF)],
        pltpu.VMEM_SHARED((N2, D), jnp.float32),
        [pltpu.SemaphoreType.DMA for _ in range(NBUF)],
        [pltpu.SemaphoreType.DMA for _ in range(NBUF)],
    ],
)
def _row_pass(src_hbm, dst_hbm, coeff_hbm, xl_hbm, acc_out,
              srcv, dstv, cfv, rows, acc_sh, sem_g, sem_s):
    c = lax.axis_index("c")
    s = lax.axis_index("s")

    def zrow(j, _):
        for q in range(D // 16):
            rows[0][j, pl.ds(q * 16, 16)] = jnp.zeros((16,), jnp.float32)
        return 0
    lax.fori_loop(0, K, zrow, 0)
    for z in range(STRIPE // K):
        pltpu.sync_copy(rows[0], acc_sh.at[pl.ds(s * STRIPE + z * K, K)])
    plsc.subcore_barrier()

    base0 = c * (EP // NC) + s * ET

    def idx_copy(i, b):
        base = base0 + i * K
        pltpu.sync_copy(src_hbm.at[pl.ds(base, K)], srcv[b])
        pltpu.sync_copy(coeff_hbm.at[pl.ds(base, K)], cfv[b])
        pltpu.sync_copy(dst_hbm.at[pl.ds(base, K)], dstv[b])

    idx_copy(0, 0)
    idx_copy(1, 1)
    pltpu.async_copy(xl_hbm.at[srcv[0]], rows[0], sem_g[0])
    pltpu.async_copy(xl_hbm.at[srcv[1]], rows[1], sem_g[1])

    def outer(i0, _):
        for bb in range(NBUF):
            b = bb
            b1 = (bb + 1) % NBUF
            b2 = (bb + 2) % NBUF
            i = i0 * NBUF + bb
            pltpu.make_async_copy(xl_hbm.at[srcv[b]], rows[b],
                                  sem_g[b]).wait()

            def scale(j, _):
                cb = plsc.load_gather(cfv[b], [jnp.full((16,), j, jnp.int32)])
                for q in range(D // 16):
                    sl = pl.ds(q * 16, 16)
                    rows[b][j, sl] = rows[b][j, sl] * cb
                return 0
            lax.fori_loop(0, K, scale, 0, unroll=4)
            pltpu.async_copy(rows[b], acc_sh.at[dstv[b]], sem_s[b], add=True)
            bm1 = (bb - 1) % NBUF

            def wait_prev():
                pltpu.make_async_copy(rows[bm1], acc_sh.at[dstv[bm1]],
                                      sem_s[bm1]).wait()
            if bb == 0:
                @pl.when(i0 > 0)
                def _():
                    wait_prev()
            else:
                wait_prev()

            def issue_next():
                idx_copy(i + 2, b2)
                pltpu.async_copy(xl_hbm.at[srcv[b2]], rows[b2], sem_g[b2])
            if bb < 2:
                issue_next()
            else:
                @pl.when(i0 < NCHUNK // NBUF - 1)
                def _():
                    issue_next()
        return 0
    lax.fori_loop(0, NCHUNK // NBUF, outer, 0)
    pltpu.make_async_copy(rows[(NCHUNK - 1) % NBUF],
                          acc_sh.at[dstv[(NCHUNK - 1) % NBUF]],
                          sem_s[(NCHUNK - 1) % NBUF]).wait()
    plsc.subcore_barrier()
    pltpu.sync_copy(acc_sh.at[pl.ds(s * STRIPE, STRIPE)],
                    acc_out.at[c, pl.ds(s * STRIPE, STRIPE)])


def _mm_bias(xin, w, b):
    m = xin.shape[0]
    bm = 1000

    def body(x_ref, w_ref, b_ref, o_ref):
        o_ref[...] = jnp.dot(x_ref[...], w_ref[...],
                             preferred_element_type=jnp.float32) + b_ref[...]

    return pl.pallas_call(
        body,
        grid=(m // bm,),
        in_specs=[
            pl.BlockSpec((bm, D), lambda i: (i, 0)),
            pl.BlockSpec((D, D), lambda i: (0, 0)),
            pl.BlockSpec((1, D), lambda i: (0, 0)),
        ],
        out_specs=pl.BlockSpec((bm, D), lambda i: (i, 0)),
        out_shape=jax.ShapeDtypeStruct((m, D), jnp.float32),
    )(xin, w, b.reshape(1, D))


def _post(acc0, acc1, selfc, xl, w, relu):
    m = xl.shape[0]
    bm = 1000

    def body(a0_ref, a1_ref, sc_ref, x_ref, w_ref, o_ref):
        aggr = a0_ref[...] + a1_ref[...] + sc_ref[...] * x_ref[...]
        o = jnp.dot(aggr, w_ref[...], preferred_element_type=jnp.float32)
        if relu:
            o = jnp.maximum(o, jnp.float32(0.0))
        o_ref[...] = o

    return pl.pallas_call(
        body,
        grid=(m // bm,),
        in_specs=[
            pl.BlockSpec((bm, D), lambda i: (i, 0)),
            pl.BlockSpec((bm, D), lambda i: (i, 0)),
            pl.BlockSpec((bm, 1), lambda i: (i, 0)),
            pl.BlockSpec((bm, D), lambda i: (i, 0)),
            pl.BlockSpec((D, D), lambda i: (0, 0)),
        ],
        out_specs=pl.BlockSpec((bm, D), lambda i: (i, 0)),
        out_shape=jax.ShapeDtypeStruct((m, D), jnp.float32),
    )(acc0, acc1, selfc, xl, w)


def kernel(x, edge_index, edge_weight, lin1_w, lin1_b, w1, lin2_w, lin2_b, w2):
    src = edge_index[0]
    dst = edge_index[1]
    pad = EP - E
    zpad = jnp.zeros((pad,), jnp.int32)
    src_p = jnp.concatenate([src, zpad])
    dst_p = jnp.concatenate([dst, zpad])
    ew_p = jnp.concatenate([edge_weight,
                            jnp.full((pad,), -100.0, jnp.float32)])

    dpart, cpart = _scalar_pass(dst_p, ew_p)
    coeff, selfc = _coeff_pass(dst_p, ew_p, dpart, cpart)
    selfc = selfc[:N].reshape(N, 1)

    xl1 = _mm_bias(x, lin1_w, lin1_b)
    acc1 = _row_pass(src_p, dst_p, coeff, xl1)
    h = _post(acc1[0, :N], acc1[1, :N], selfc, xl1, w1, relu=True)

    xl2 = _mm_bias(h, lin2_w, lin2_b)
    acc2 = _row_pass(src_p, dst_p, coeff, xl2)
    out = _post(acc2[0, :N], acc2[1, :N], selfc, xl2, w2, relu=False)
    return out


# speedup vs baseline: 7.3839x; 1.0225x over previous
import functools

import numpy as np
import jax
import jax.numpy as jnp
from jax import lax
from jax.experimental import pallas as pl
from jax.experimental.pallas import tpu as pltpu
from jax.experimental.pallas import tpu_sc as plsc

N = 10000
D = 128
E = 320000
NC = 2
NS = 16
NW = NC * NS
EP = 327680
ET = EP // NW
K = 64
NCHUNK = ET // K
NBUF = 4
KS = 512
NCHS = ET // KS
N2 = 10240
STRIPE = N2 // NS
EXP1 = float(np.exp(np.float32(1.0)))

_MESH = dict(core_axis_name="c", subcore_axis_name="s")


def _iota16():
    return lax.iota(jnp.int32, 16)


@functools.partial(
    pl.kernel,
    mesh=plsc.VectorSubcoreMesh(**_MESH),
    compiler_params=pltpu.CompilerParams(needs_layout_passes=False),
    out_type=(
        jax.ShapeDtypeStruct((NC, N2), jnp.float32),
        jax.ShapeDtypeStruct((NC, N2), jnp.float32),
    ),
    scratch_types=[
        [pltpu.VMEM((KS * 2,), jnp.int32) for _ in range(3)],
        pltpu.VMEM((KS,), jnp.int32),
        pltpu.VMEM((KS,), jnp.float32),
        pltpu.VMEM((KS,), jnp.float32),
        pltpu.VMEM((STRIPE,), jnp.float32),
        pltpu.VMEM_SHARED((N2,), jnp.float32),
        pltpu.VMEM_SHARED((N2,), jnp.float32),
        [pltpu.SemaphoreType.DMA for _ in range(3)],
    ],
)
def _scalar_pass(ed2_hbm, d_out, c_out,
                 eb, dstb, valsb, onesb, zv, d_sh, c_sh, sem_i):
    c = lax.axis_index("c")
    s = lax.axis_index("s")

    def zbody(j, _):
        zv[pl.ds(j * 16, 16)] = jnp.zeros((16,), jnp.float32)
        return 0
    lax.fori_loop(0, STRIPE // 16, zbody, 0)
    pltpu.sync_copy(zv, d_sh.at[pl.ds(s * STRIPE, STRIPE)])
    pltpu.sync_copy(zv, c_sh.at[pl.ds(s * STRIPE, STRIPE)])
    plsc.subcore_barrier()

    base0 = (c * (EP // NC) + s * ET) * 2

    def istart(i, b):
        pltpu.async_copy(ed2_hbm.at[pl.ds(base0 + i * KS * 2, KS * 2)],
                         eb[b], sem_i[b])

    def iwait(i, b):
        pltpu.make_async_copy(ed2_hbm.at[pl.ds(base0 + i * KS * 2, KS * 2)],
                              eb[b], sem_i[b]).wait()

    istart(0, 0)
    istart(1, 1)
    iot2 = _iota16() * 2
    for i in range(NCHS):
        b = i % 3
        iwait(i, b)

        def compute(g, _):
            sl = pl.ds(g * 16, 16)
            idx = iot2 + g * 32
            dstb[sl] = plsc.load_gather(eb[b], [idx])
            w16 = plsc.bitcast(plsc.load_gather(eb[b], [idx + 1]),
                               jnp.float32)
            valsb[sl] = jnp.exp(w16)
            onesb[sl] = jnp.where(w16 > jnp.float32(-50.0),
                                  jnp.float32(1.0), jnp.float32(0.0))
            return 0
        lax.fori_loop(0, KS // 16, compute, 0, unroll=2)
        pltpu.sync_copy(valsb, d_sh.at[dstb], add=True)
        pltpu.sync_copy(onesb, c_sh.at[dstb], add=True)
        if i + 2 < NCHS:
            istart(i + 2, (i + 2) % 3)
    plsc.subcore_barrier()
    pltpu.sync_copy(d_sh.at[pl.ds(s * STRIPE, STRIPE)],
                    d_out.at[c, pl.ds(s * STRIPE, STRIPE)])
    pltpu.sync_copy(c_sh.at[pl.ds(s * STRIPE, STRIPE)],
                    c_out.at[c, pl.ds(s * STRIPE, STRIPE)])


@functools.partial(
    pl.kernel,
    mesh=plsc.VectorSubcoreMesh(**_MESH),
    compiler_params=pltpu.CompilerParams(needs_layout_passes=False),
    out_type=(
        jax.ShapeDtypeStruct((EP,), jnp.float32),
        jax.ShapeDtypeStruct((N2,), jnp.float32),
    ),
    scratch_types=[
        pltpu.VMEM((N2,), jnp.float32),
        pltpu.VMEM((N2,), jnp.float32),
        pltpu.VMEM((N2,), jnp.float32),
        pltpu.VMEM((N2 // NW,), jnp.float32),
        [pltpu.VMEM((KS * 2,), jnp.int32) for _ in range(3)],
        pltpu.VMEM((KS,), jnp.float32),
        [pltpu.SemaphoreType.DMA for _ in range(3)],
    ],
)
def _coeff_pass(ed2_hbm, dpart, cpart, coeff_out, selfc_out,
                dloc, cloc, tmp, selfv, eb, outv, sem_i):
    c = lax.axis_index("c")
    s = lax.axis_index("s")
    wid = s * NC + c

    base0 = (c * (EP // NC) + s * ET) * 2

    def istart(i, b):
        pltpu.async_copy(ed2_hbm.at[pl.ds(base0 + i * KS * 2, KS * 2)],
                         eb[b], sem_i[b])

    def iwait(i, b):
        pltpu.make_async_copy(ed2_hbm.at[pl.ds(base0 + i * KS * 2, KS * 2)],
                              eb[b], sem_i[b]).wait()

    istart(0, 0)
    istart(1, 1)

    pltpu.sync_copy(dpart.at[0], dloc)
    pltpu.sync_copy(dpart.at[1], tmp)

    def comb_d(j, _):
        sl = pl.ds(j * 16, 16)
        dloc[sl] = dloc[sl] + tmp[sl] + jnp.float32(EXP1)
        return 0
    lax.fori_loop(0, N2 // 16, comb_d, 0, unroll=4)

    pltpu.sync_copy(cpart.at[0], cloc)
    pltpu.sync_copy(cpart.at[1], tmp)

    def comb_c(j, _):
        sl = pl.ds(j * 16, 16)
        cloc[sl] = cloc[sl] + tmp[sl] + jnp.float32(1.0)
        return 0
    lax.fori_loop(0, N2 // 16, comb_c, 0, unroll=4)

    nper = N2 // NW
    nbase = wid * nper

    def selfc_body(j, _):
        sl = pl.ds(j * 16, 16)
        gl = pl.ds(nbase + j * 16, 16)
        selfv[sl] = jnp.float32(EXP1) / (dloc[gl] * cloc[gl])
        return 0
    lax.fori_loop(0, nper // 16, selfc_body, 0)
    pltpu.sync_copy(selfv, selfc_out.at[pl.ds(nbase, nper)])

    cbase0 = c * (EP // NC) + s * ET
    iot2 = _iota16() * 2
    for i in range(NCHS):
        b = i % 3
        iwait(i, b)

        def compute(g, _):
            sl = pl.ds(g * 16, 16)
            idx = iot2 + g * 32
            dst16 = plsc.load_gather(eb[b], [idx])
            w16 = plsc.bitcast(plsc.load_gather(eb[b], [idx + 1]),
                               jnp.float32)
            dd = plsc.load_gather(dloc, [dst16])
            cc = plsc.load_gather(cloc, [dst16])
            outv[sl] = jnp.exp(w16) / (dd * cc)
            return 0
        lax.fori_loop(0, KS // 16, compute, 0, unroll=2)
        pltpu.sync_copy(outv, coeff_out.at[pl.ds(cbase0 + i * KS, KS)])
        if i + 2 < NCHS:
            istart(i + 2, (i + 2) % 3)


@functools.partial(
    pl.kernel,
    mesh=plsc.VectorSubcoreMesh(**_MESH),
    compiler_params=pltpu.CompilerParams(needs_layout_passes=False),
    out_type=jax.ShapeDtypeStruct((NC, N2, D), jnp.float32),
    scratch_types=[
        [pltpu.VMEM((K * 3,), jnp.int32) for _ in range(NBUF)],
        [pltpu.VMEM((K,), jnp.int32) for _ in range(NBUF)],
        [pltpu.VMEM((K,), jnp.int32) for _ in range(NBUF)],
        [pltpu.VMEM((K, D), jnp.float32) for _ in range(NBUF)],
        pltpu.VMEM_SHARED((N2, D), jnp.float32),
        [pltpu.SemaphoreType.DMA for _ in range(NBUF)],
        [pltpu.SemaphoreType.DMA for _ in range(NBUF)],
        [pltpu.SemaphoreType.DMA for _ in range(NBUF)],
    ],
)
def _row_pass(ed3_hbm, xl_hbm, acc_out,
              ebuf, srcv, dstv, rows, acc_sh, sem_i, sem_g, sem_s):
    c = lax.axis_index("c")
    s = lax.axis_index("s")

    def zrow(j, _):
        for q in range(D // 16):
            rows[0][j, pl.ds(q * 16, 16)] = jnp.zeros((16,), jnp.float32)
        return 0
    lax.fori_loop(0, K, zrow, 0)
    for z in range(STRIPE // K):
        pltpu.sync_copy(rows[0], acc_sh.at[pl.ds(s * STRIPE + z * K, K)])
    plsc.subcore_barrier()

    base0 = (c * (EP // NC) + s * ET) * 3
    iot3 = _iota16() * 3

    def istart(i, b):
        pltpu.async_copy(ed3_hbm.at[pl.ds(base0 + i * K * 3, K * 3)],
                         ebuf[b], sem_i[b])

    def iwait(i, b):
        pltpu.make_async_copy(ed3_hbm.at[pl.ds(base0 + i * K * 3, K * 3)],
                              ebuf[b], sem_i[b]).wait()

    def deint(b):
        for g in range(K // 16):
            idx = iot3 + g * 48
            sl = pl.ds(g * 16, 16)
            srcv[b][sl] = plsc.load_gather(ebuf[b], [idx])
            dstv[b][sl] = plsc.load_gather(ebuf[b], [idx + 1])

    istart(0, 0)
    istart(1, 1)
    istart(2, 2)
    iwait(0, 0)
    deint(0)
    pltpu.async_copy(xl_hbm.at[srcv[0]], rows[0], sem_g[0])
    iwait(1, 1)
    deint(1)
    pltpu.async_copy(xl_hbm.at[srcv[1]], rows[1], sem_g[1])

    def outer(i0, _):
        for bb in range(NBUF):
            b = bb
            b2 = (bb + 2) % NBUF
            b3 = (bb + 3) % NBUF
            bm1 = (bb - 1) % NBUF
            i = i0 * NBUF + bb
            pltpu.make_async_copy(xl_hbm.at[srcv[b]], rows[b],
                                  sem_g[b]).wait()

            def scale(j, _):
                cb = plsc.bitcast(plsc.load_gather(
                    ebuf[b], [jnp.full((16,), j * 3 + 2, jnp.int32)]),
                    jnp.float32)
                for q in range(D // 16):
                    sl = pl.ds(q * 16, 16)
                    rows[b][j, sl] = rows[b][j, sl] * cb
                return 0
            lax.fori_loop(0, K, scale, 0, unroll=4)
            pltpu.async_copy(rows[b], acc_sh.at[dstv[b]], sem_s[b], add=True)

            def wait_prev():
                pltpu.make_async_copy(rows[bm1], acc_sh.at[dstv[bm1]],
                                      sem_s[bm1]).wait()
            if bb == 0:
                @pl.when(i0 > 0)
                def _():
                    wait_prev()
            else:
                wait_prev()

            def issue_gather():
                iwait(i + 2, b2)
                deint(b2)
                pltpu.async_copy(xl_hbm.at[srcv[b2]], rows[b2], sem_g[b2])
            if bb < 2:
                issue_gather()
            else:
                @pl.when(i0 < NCHUNK // NBUF - 1)
                def _():
                    issue_gather()

            def issue_rec():
                istart(i + 3, b3)
            if bb == 0:
                issue_rec()
            else:
                @pl.when(i0 < NCHUNK // NBUF - 1)
                def _():
                    issue_rec()
        return 0
    lax.fori_loop(0, NCHUNK // NBUF, outer, 0)
    pltpu.make_async_copy(rows[(NCHUNK - 1) % NBUF],
                          acc_sh.at[dstv[(NCHUNK - 1) % NBUF]],
                          sem_s[(NCHUNK - 1) % NBUF]).wait()
    plsc.subcore_barrier()
    pltpu.sync_copy(acc_sh.at[pl.ds(s * STRIPE, STRIPE)],
                    acc_out.at[c, pl.ds(s * STRIPE, STRIPE)])


def _mm_bias(xin, w, b):
    m = xin.shape[0]
    bm = 1000

    def body(x_ref, w_ref, b_ref, o_ref):
        o_ref[...] = jnp.dot(x_ref[...], w_ref[...],
                             preferred_element_type=jnp.float32) + b_ref[...]

    return pl.pallas_call(
        body,
        grid=(m // bm,),
        in_specs=[
            pl.BlockSpec((bm, D), lambda i: (i, 0)),
            pl.BlockSpec((D, D), lambda i: (0, 0)),
            pl.BlockSpec((1, D), lambda i: (0, 0)),
        ],
        out_specs=pl.BlockSpec((bm, D), lambda i: (i, 0)),
        out_shape=jax.ShapeDtypeStruct((m, D), jnp.float32),
    )(xin, w, b.reshape(1, D))


def _post(acc0, acc1, selfc, xl, w, relu):
    m = xl.shape[0]
    bm = 1000

    def body(a0_ref, a1_ref, sc_ref, x_ref, w_ref, o_ref):
        aggr = a0_ref[...] + a1_ref[...] + sc_ref[...] * x_ref[...]
        o = jnp.dot(aggr, w_ref[...], preferred_element_type=jnp.float32)
        if relu:
            o = jnp.maximum(o, jnp.float32(0.0))
        o_ref[...] = o

    return pl.pallas_call(
        body,
        grid=(m // bm,),
        in_specs=[
            pl.BlockSpec((bm, D), lambda i: (i, 0)),
            pl.BlockSpec((bm, D), lambda i: (i, 0)),
            pl.BlockSpec((bm, 1), lambda i: (i, 0)),
            pl.BlockSpec((bm, D), lambda i: (i, 0)),
            pl.BlockSpec((D, D), lambda i: (0, 0)),
        ],
        out_specs=pl.BlockSpec((bm, D), lambda i: (i, 0)),
        out_shape=jax.ShapeDtypeStruct((m, D), jnp.float32),
    )(acc0, acc1, selfc, xl, w)


def kernel(x, edge_index, edge_weight, lin1_w, lin1_b, w1, lin2_w, lin2_b, w2):
    src = edge_index[0]
    dst = edge_index[1]
    pad = EP - E
    zpad = jnp.zeros((pad,), jnp.int32)
    src_p = jnp.concatenate([src, zpad])
    dst_p = jnp.concatenate([dst, zpad])
    ew_p = jnp.concatenate([edge_weight,
                            jnp.full((pad,), -100.0, jnp.float32)])
    ewb = lax.bitcast_convert_type(ew_p, jnp.int32)
    ed2 = jnp.stack([dst_p, ewb], axis=1).reshape(-1)

    dpart, cpart = _scalar_pass(ed2)
    coeff, selfc = _coeff_pass(ed2, dpart, cpart)
    selfc = selfc[:N].reshape(N, 1)
    ed3 = jnp.stack(
        [src_p, dst_p, lax.bitcast_convert_type(coeff, jnp.int32)],
        axis=1).reshape(-1)

    xl1 = _mm_bias(x, lin1_w, lin1_b)
    acc1 = _row_pass(ed3, xl1)
    h = _post(acc1[0, :N], acc1[1, :N], selfc, xl1, w1, relu=True)

    xl2 = _mm_bias(h, lin2_w, lin2_b)
    acc2 = _row_pass(ed3, xl2)
    out = _post(acc2[0, :N], acc2[1, :N], selfc, xl2, w2, relu=False)
    return out
